# baseline (speedup 1.0000x reference)
import functools

import jax
import jax.numpy as jnp
from jax import lax
from jax.experimental import pallas as pl
from jax.experimental.pallas import tpu as pltpu
from jax.experimental.pallas import tpu_sc as plsc

B = 512
ROIS = 90
RP = 96
LP = 128
EPG = 2880
E = B * EPG
DIM = 96
NH = 2
HD = 48
ATTEN_S = 20
ZW = ROIS * DIM
NC = 2
NS = 16
NW = NC * NS
GPW = B // NW


def _adj_body(pidx_hbm, ew_hbm, out_hbm,
              i0, i1, w0, w1, acc0, acc1, se0, se1, so0, so1):
    wid = lax.axis_index("s") * NC + lax.axis_index("c")
    g0 = wid * GPW

    def load_edges(g, ib, wb, se):
        sl = pl.ds(g * EPG, EPG)
        pltpu.async_copy(pidx_hbm.at[sl], ib, se)
        pltpu.async_copy(ew_hbm.at[sl], wb, se)

    def wait_edges(ib, wb, se):
        sl = pl.ds(0, EPG)
        pltpu.make_async_copy(pidx_hbm.at[sl], ib, se).wait()
        pltpu.make_async_copy(ew_hbm.at[sl], wb, se).wait()

    load_edges(g0, i0, w0, se0)
    load_edges(g0 + 1, i1, w1, se1)
    lanes = lax.iota(jnp.int32, 16)
    ones16 = jnp.ones((16,), jnp.float32)

    def process(g, p, ib, wb, acc, se, so):
        @pl.when(p >= 2)
        def _():
            pltpu.make_async_copy(acc, out_hbm.at[0], so).wait()

        def zrow(r, c):
            for j in range(LP // 16):
                acc[r, pl.ds(j * 16, 16)] = jnp.zeros((16,), jnp.float32)
            return c

        lax.fori_loop(0, RP, zrow, 0, unroll=4)
        for j in range(RP // 16):
            dv = lanes + (j * 16)
            plsc.addupdate_scatter(acc, [dv, dv], ones16)
        wait_edges(ib, wb, se)
        off = g * (ROIS * (LP + 1))

        def edge(j, c):
            iv = ib[pl.ds(j * 16, 16)] - off
            wv = wb[pl.ds(j * 16, 16)]
            plsc.addupdate_scatter(acc, [iv >> 7, iv & 127], wv)
            return c

        lax.fori_loop(0, EPG // 16, edge, 0, unroll=8)
        pltpu.async_copy(acc, out_hbm.at[g], so)

        @pl.when(p < GPW - 2)
        def _():
            load_edges(g + 2, ib, wb, se)

    def pair(p, c):
        gA = g0 + 2 * p
        process(gA, 2 * p, i0, w0, acc0, se0, so0)
        process(gA + 1, 2 * p + 1, i1, w1, acc1, se1, so1)
        return c

    lax.fori_loop(0, GPW // 2, pair, 0)
    pltpu.make_async_copy(acc0, out_hbm.at[0], so0).wait()
    pltpu.make_async_copy(acc1, out_hbm.at[0], so1).wait()


def _build_adjacency(pidx, ew):
    mesh = plsc.VectorSubcoreMesh(core_axis_name="c", subcore_axis_name="s")
    k = functools.partial(
        pl.kernel,
        mesh=mesh,
        compiler_params=pltpu.CompilerParams(needs_layout_passes=False),
        out_type=jax.ShapeDtypeStruct((B, RP, LP), jnp.float32),
        scratch_types=[
            pltpu.VMEM((EPG,), jnp.int32),
            pltpu.VMEM((EPG,), jnp.int32),
            pltpu.VMEM((EPG,), jnp.float32),
            pltpu.VMEM((EPG,), jnp.float32),
            pltpu.VMEM((RP, LP), jnp.float32),
            pltpu.VMEM((RP, LP), jnp.float32),
            pltpu.SemaphoreType.DMA,
            pltpu.SemaphoreType.DMA,
            pltpu.SemaphoreType.DMA,
            pltpu.SemaphoreType.DMA,
        ],
    )(_adj_body)
    return k(pidx, ew)


def _mm(a, w):
    return lax.dot_general(a, w, (((a.ndim - 1,), (0,)), ((), ())),
                           preferred_element_type=jnp.float32)


def _bmm(a, b):
    return lax.dot_general(a, b, (((2,), (1,)), ((0,), (0,))),
                           preferred_element_type=jnp.float32)


def _gnn_body(w_ref, x_ref, snps_ref,
              w1_ref, b1_ref, w2_ref, b2_ref, w3_ref, b3_ref,
              we_ref, be_ref, wd_ref, bd_ref, wa_ref, ba_ref,
              wq0_ref, bq0_ref, wq1_ref, bq1_ref,
              wk0_ref, bk0_ref, wk1_ref, bk1_ref,
              wv0_ref, bv0_ref, wv1_ref, bv1_ref,
              wo0_ref, wo1_ref, bo_ref,
              wc1a_ref, wc1b_ref, bc1_ref, wc2_ref, bc2_ref,
              wu1a_ref, wu1b_ref, bu1_ref, wu2_ref, bu2_ref,
              lc_ref, lu_ref, xhat_ref, z_ref):
    G = w_ref.shape[0]
    A = w_ref[...][:, :, :RP]
    onesRP = jnp.ones((RP, 1), jnp.float32)
    deg3 = _mm(A, onesRP)
    dinv3 = jnp.where(deg3 > 0, lax.rsqrt(deg3), 0.0)

    xg = x_ref[...]
    h0 = xg[:, :, None] * w1_ref[...][0][None, None, :]
    x1 = jnp.maximum(dinv3 * _bmm(A, dinv3 * h0) + b1_ref[...][0], 0.0)
    x2 = jnp.maximum(dinv3 * _bmm(A, dinv3 * _mm(x1, w2_ref[...]))
                     + b2_ref[...][0], 0.0)
    x3 = jnp.maximum(dinv3 * _bmm(A, dinv3 * _mm(x2, w3_ref[...]))
                     + b3_ref[...][0], 0.0)
    xcat = jnp.concatenate([x1, x2, x3], axis=2)

    snps = snps_ref[...]
    latent = jnp.tanh(_mm(snps, we_ref[...]) + be_ref[...][0])
    xhat_ref[...] = _mm(latent, wd_ref[...]) + bd_ref[...][0]

    ao = (_mm(snps, wa_ref[...]) + ba_ref[...][0]).reshape(G, ATTEN_S, DIM)

    scale = 1.0 / (HD ** 0.5)
    onesS = jnp.ones((ATTEN_S, 1), jnp.float32)
    head_w = ((wq0_ref, bq0_ref, wk0_ref, bk0_ref, wv0_ref, bv0_ref),
              (wq1_ref, bq1_ref, wk1_ref, bk1_ref, wv1_ref, bv1_ref))
    outs = []
    for wq, bq, wk, bk, wv, bv in head_w:
        qh = _mm(xcat, wq[...]) + bq[...][0]
        kh = _mm(ao, wk[...]) + bk[...][0]
        vh = _mm(ao, wv[...]) + bv[...][0]
        s = lax.dot_general(qh, kh, (((2,), (2,)), ((0,), (0,))),
                            preferred_element_type=jnp.float32) * scale
        m = jnp.max(s, axis=2, keepdims=True)
        p = jnp.exp(s - m)
        a = p * (1.0 / _mm(p, onesS))
        outs.append(_bmm(a, vh))
    attn = jnp.maximum(_mm(outs[0], wo0_ref[...]) + _mm(outs[1], wo1_ref[...])
                       + bo_ref[...][0], 0.0)
    y = ((xcat + attn) * 0.5)[:, :ROIS, :]
    yflat = y.reshape(G, ZW)
    z_ref[:, :ZW] = yflat
    z_ref[:, ZW:] = latent

    def lsm(v):
        mm_ = jnp.max(v, axis=-1, keepdims=True)
        ee = v - mm_
        return ee - jnp.log(jnp.sum(jnp.exp(ee), axis=-1, keepdims=True))

    hc = jnp.maximum(_mm(yflat, wc1a_ref[...]) + _mm(latent, wc1b_ref[...])
                     + bc1_ref[...][0], 0.0)
    lc_ref[...] = lsm(_mm(hc, wc2_ref[...]) + bc2_ref[...][0])
    hu = jnp.maximum(_mm(yflat, wu1a_ref[...]) + _mm(latent, wu1b_ref[...])
                     + bu1_ref[...][0], 0.0)
    lu_ref[...] = lsm(_mm(hu, wu2_ref[...]) + bu2_ref[...][0])


def _const_spec(arr):
    nd = arr.ndim
    return pl.BlockSpec(arr.shape, lambda i, _n=nd: (0,) * _n)


def _gnn_call(wmat3, xg96, snps, weights, G):
    grid = (B // G,)
    in_specs = [
        pl.BlockSpec((G, RP, LP), lambda i: (i, 0, 0)),
        pl.BlockSpec((G, RP), lambda i: (i, 0)),
        pl.BlockSpec((G, snps.shape[1]), lambda i: (i, 0)),
    ] + [_const_spec(w) for w in weights]
    out_specs = [
        pl.BlockSpec((G, 3), lambda i: (i, 0)),
        pl.BlockSpec((G, 2), lambda i: (i, 0)),
        pl.BlockSpec((G, 54), lambda i: (i, 0)),
        pl.BlockSpec((G, ZW + 64), lambda i: (i, 0)),
    ]
    out_shape = [
        jax.ShapeDtypeStruct((B, 3), jnp.float32),
        jax.ShapeDtypeStruct((B, 2), jnp.float32),
        jax.ShapeDtypeStruct((B, 54), jnp.float32),
        jax.ShapeDtypeStruct((B, ZW + 64), jnp.float32),
    ]
    return pl.pallas_call(
        _gnn_body, grid=grid, in_specs=in_specs, out_specs=out_specs,
        out_shape=out_shape,
    )(wmat3, xg96, snps, *weights)


def kernel(x, edge_index, edge_weight, batch, snps_feat, temperature,
           W1, b1, W2, b2, W3, b3, We, be, Wd, bd, Wa, ba, Wq, bq,
           Wk, bk, Wv, bv, Wo, bo, Wc1, bc1, Wc2, bc2, Wu1, bu1, Wu2, bu2):
    pidx = (edge_index * jnp.array([[1], [LP]], dtype=jnp.int32)).sum(axis=0)
    wmat3 = _build_adjacency(pidx, edge_weight)

    xg96 = jnp.pad(x.reshape(B, ROIS), ((0, 0), (0, RP - ROIS)))
    weights = [
        W1, b1.reshape(1, -1), W2, b2.reshape(1, -1), W3, b3.reshape(1, -1),
        We, be.reshape(1, -1), Wd, bd.reshape(1, -1), Wa, ba.reshape(1, -1),
        Wq[:, :HD], bq[:HD].reshape(1, -1), Wq[:, HD:], bq[HD:].reshape(1, -1),
        Wk[:, :HD], bk[:HD].reshape(1, -1), Wk[:, HD:], bk[HD:].reshape(1, -1),
        Wv[:, :HD], bv[:HD].reshape(1, -1), Wv[:, HD:], bv[HD:].reshape(1, -1),
        Wo[:HD], Wo[HD:], bo.reshape(1, -1),
        Wc1[:ZW], Wc1[ZW:], bc1.reshape(1, -1), Wc2, bc2.reshape(1, -1),
        Wu1[:ZW], Wu1[ZW:], bu1.reshape(1, -1), Wu2, bu2.reshape(1, -1),
    ]
    log_c, log_u, x_hat, out_z = _gnn_call(wmat3, xg96, snps_feat,
                                           weights, 64)
    return (log_c, log_u, x_hat, out_z)


# speedup vs baseline: 1.2386x; 1.2386x over previous
import functools

import jax
import jax.numpy as jnp
from jax import lax
from jax.experimental import pallas as pl
from jax.experimental.pallas import tpu as pltpu
from jax.experimental.pallas import tpu_sc as plsc

B = 512
ROIS = 90
RP = 96
EPG = 2880
E = B * EPG
DIM = 96
NH = 2
HD = 48
ATTEN_S = 20
ZW = ROIS * DIM
NC = 2
NS = 16
NW = NC * NS
GPW = B // NW


def _adj_body(row_hbm, col_hbm, ew_hbm, out_hbm,
              r0, r1, c0, c1, w0, w1, acc0, acc1, se0, se1, so0, so1):
    wid = lax.axis_index("s") * NC + lax.axis_index("c")
    g0 = wid * GPW

    def load_edges(g, rb, cb, wb, se):
        sl = pl.ds(g * EPG, EPG)
        pltpu.async_copy(row_hbm.at[sl], rb, se)
        pltpu.async_copy(col_hbm.at[sl], cb, se)
        pltpu.async_copy(ew_hbm.at[sl], wb, se)

    def wait_edges(rb, cb, wb, se):
        sl = pl.ds(0, EPG)
        pltpu.make_async_copy(row_hbm.at[sl], rb, se).wait()
        pltpu.make_async_copy(col_hbm.at[sl], cb, se).wait()
        pltpu.make_async_copy(ew_hbm.at[sl], wb, se).wait()

    load_edges(g0, r0, c0, w0, se0)
    load_edges(g0 + 1, r1, c1, w1, se1)
    lanes = lax.iota(jnp.int32, 16)
    ones16 = jnp.ones((16,), jnp.float32)

    def process(g, p, rb, cb, wb, acc, se, so):
        @pl.when(p >= 2)
        def _():
            pltpu.make_async_copy(acc, out_hbm.at[0], so).wait()

        def zrow(r, c):
            for j in range(RP // 16):
                acc[r, pl.ds(j * 16, 16)] = jnp.zeros((16,), jnp.float32)
            return c

        lax.fori_loop(0, RP, zrow, 0, unroll=4)
        for j in range(RP // 16):
            dv = lanes + (j * 16)
            plsc.addupdate_scatter(acc, [dv, dv], ones16)
        wait_edges(rb, cb, wb, se)
        base = g * ROIS

        def edge(j, c):
            rv = rb[pl.ds(j * 16, 16)] - base
            cv = cb[pl.ds(j * 16, 16)] - base
            wv = wb[pl.ds(j * 16, 16)]
            plsc.addupdate_scatter(acc, [cv, rv], wv)
            return c

        lax.fori_loop(0, EPG // 16, edge, 0, unroll=4)
        pltpu.async_copy(acc, out_hbm.at[g], so)

        @pl.when(p < GPW - 2)
        def _():
            load_edges(g + 2, rb, cb, wb, se)

    def pair(p, c):
        gA = g0 + 2 * p
        process(gA, 2 * p, r0, c0, w0, acc0, se0, so0)
        process(gA + 1, 2 * p + 1, r1, c1, w1, acc1, se1, so1)
        return c

    lax.fori_loop(0, GPW // 2, pair, 0)
    pltpu.make_async_copy(acc0, out_hbm.at[0], so0).wait()
    pltpu.make_async_copy(acc1, out_hbm.at[0], so1).wait()


def _build_adjacency(row, col, ew):
    mesh = plsc.VectorSubcoreMesh(core_axis_name="c", subcore_axis_name="s")
    k = functools.partial(
        pl.kernel,
        mesh=mesh,
        compiler_params=pltpu.CompilerParams(needs_layout_passes=False),
        out_type=jax.ShapeDtypeStruct((B, RP, RP), jnp.float32),
        scratch_types=[
            pltpu.VMEM((EPG,), jnp.int32),
            pltpu.VMEM((EPG,), jnp.int32),
            pltpu.VMEM((EPG,), jnp.int32),
            pltpu.VMEM((EPG,), jnp.int32),
            pltpu.VMEM((EPG,), jnp.float32),
            pltpu.VMEM((EPG,), jnp.float32),
            pltpu.VMEM((RP, RP), jnp.float32),
            pltpu.VMEM((RP, RP), jnp.float32),
            pltpu.SemaphoreType.DMA,
            pltpu.SemaphoreType.DMA,
            pltpu.SemaphoreType.DMA,
            pltpu.SemaphoreType.DMA,
        ],
    )(_adj_body)
    return k(row, col, ew)


def _mm(a, w):
    return lax.dot_general(a, w, (((a.ndim - 1,), (0,)), ((), ())),
                           preferred_element_type=jnp.float32)


def _bmm(a, b):
    return lax.dot_general(a, b, (((2,), (1,)), ((0,), (0,))),
                           preferred_element_type=jnp.float32)


def _gnn_body(w_ref, x_ref, snps_ref,
              w1_ref, b1_ref, w2_ref, b2_ref, w3_ref, b3_ref,
              we_ref, be_ref, wd_ref, bd_ref, wa_ref, ba_ref,
              wq0_ref, bq0_ref, wq1_ref, bq1_ref,
              wk0_ref, bk0_ref, wk1_ref, bk1_ref,
              wv0_ref, bv0_ref, wv1_ref, bv1_ref,
              wo0_ref, wo1_ref, bo_ref,
              wc1a_ref, wc1b_ref, bc1_ref, wc2_ref, bc2_ref,
              wu1a_ref, wu1b_ref, bu1_ref, wu2_ref, bu2_ref,
              lc_ref, lu_ref, xhat_ref, z_ref):
    G = w_ref.shape[0]
    A = w_ref[...]
    onesRP = jnp.ones((RP, 1), jnp.float32)
    deg3 = _mm(A, onesRP)
    dinv3 = jnp.where(deg3 > 0, lax.rsqrt(deg3), 0.0)

    xg = x_ref[...]
    h0 = xg[:, :, None] * w1_ref[...][0][None, None, :]
    x1 = jnp.maximum(dinv3 * _bmm(A, dinv3 * h0) + b1_ref[...][0], 0.0)
    x2 = jnp.maximum(dinv3 * _bmm(A, dinv3 * _mm(x1, w2_ref[...]))
                     + b2_ref[...][0], 0.0)
    x3 = jnp.maximum(dinv3 * _bmm(A, dinv3 * _mm(x2, w3_ref[...]))
                     + b3_ref[...][0], 0.0)
    xcat = jnp.concatenate([x1, x2, x3], axis=2)

    snps = snps_ref[...]
    latent = jnp.tanh(_mm(snps, we_ref[...]) + be_ref[...][0])
    xhat_ref[...] = _mm(latent, wd_ref[...]) + bd_ref[...][0]

    ao = (_mm(snps, wa_ref[...]) + ba_ref[...][0]).reshape(G, ATTEN_S, DIM)

    scale = 1.0 / (HD ** 0.5)
    onesS = jnp.ones((ATTEN_S, 1), jnp.float32)
    head_w = ((wq0_ref, bq0_ref, wk0_ref, bk0_ref, wv0_ref, bv0_ref),
              (wq1_ref, bq1_ref, wk1_ref, bk1_ref, wv1_ref, bv1_ref))
    outs = []
    for wq, bq, wk, bk, wv, bv in head_w:
        qh = _mm(xcat, wq[...]) + bq[...][0]
        kh = _mm(ao, wk[...]) + bk[...][0]
        vh = _mm(ao, wv[...]) + bv[...][0]
        s = lax.dot_general(qh, kh, (((2,), (2,)), ((0,), (0,))),
                            preferred_element_type=jnp.float32) * scale
        m = jnp.max(s, axis=2, keepdims=True)
        p = jnp.exp(s - m)
        a = p * (1.0 / _mm(p, onesS))
        outs.append(_bmm(a, vh))
    attn = jnp.maximum(_mm(outs[0], wo0_ref[...]) + _mm(outs[1], wo1_ref[...])
                       + bo_ref[...][0], 0.0)
    y = ((xcat + attn) * 0.5)[:, :ROIS, :]
    yflat = y.reshape(G, ZW)
    z_ref[:, :ZW] = yflat
    z_ref[:, ZW:] = latent

    def lsm(v):
        mm_ = jnp.max(v, axis=-1, keepdims=True)
        ee = v - mm_
        return ee - jnp.log(jnp.sum(jnp.exp(ee), axis=-1, keepdims=True))

    hc = jnp.maximum(_mm(yflat, wc1a_ref[...]) + _mm(latent, wc1b_ref[...])
                     + bc1_ref[...][0], 0.0)
    lc_ref[...] = lsm(_mm(hc, wc2_ref[...]) + bc2_ref[...][0])
    hu = jnp.maximum(_mm(yflat, wu1a_ref[...]) + _mm(latent, wu1b_ref[...])
                     + bu1_ref[...][0], 0.0)
    lu_ref[...] = lsm(_mm(hu, wu2_ref[...]) + bu2_ref[...][0])


def _const_spec(arr):
    nd = arr.ndim
    return pl.BlockSpec(arr.shape, lambda i, _n=nd: (0,) * _n)


def _gnn_call(wmat3, xg96, snps, weights, G):
    grid = (B // G,)
    in_specs = [
        pl.BlockSpec((G, RP, RP), lambda i: (i, 0, 0)),
        pl.BlockSpec((G, RP), lambda i: (i, 0)),
        pl.BlockSpec((G, snps.shape[1]), lambda i: (i, 0)),
    ] + [_const_spec(w) for w in weights]
    out_specs = [
        pl.BlockSpec((G, 3), lambda i: (i, 0)),
        pl.BlockSpec((G, 2), lambda i: (i, 0)),
        pl.BlockSpec((G, 54), lambda i: (i, 0)),
        pl.BlockSpec((G, ZW + 64), lambda i: (i, 0)),
    ]
    out_shape = [
        jax.ShapeDtypeStruct((B, 3), jnp.float32),
        jax.ShapeDtypeStruct((B, 2), jnp.float32),
        jax.ShapeDtypeStruct((B, 54), jnp.float32),
        jax.ShapeDtypeStruct((B, ZW + 64), jnp.float32),
    ]
    return pl.pallas_call(
        _gnn_body, grid=grid, in_specs=in_specs, out_specs=out_specs,
        out_shape=out_shape,
    )(wmat3, xg96, snps, *weights)


def kernel(x, edge_index, edge_weight, batch, snps_feat, temperature,
           W1, b1, W2, b2, W3, b3, We, be, Wd, bd, Wa, ba, Wq, bq,
           Wk, bk, Wv, bv, Wo, bo, Wc1, bc1, Wc2, bc2, Wu1, bu1, Wu2, bu2):
    wmat3 = _build_adjacency(edge_index[0], edge_index[1], edge_weight)

    xg96 = jnp.pad(x.reshape(B, ROIS), ((0, 0), (0, RP - ROIS)))
    weights = [
        W1, b1.reshape(1, -1), W2, b2.reshape(1, -1), W3, b3.reshape(1, -1),
        We, be.reshape(1, -1), Wd, bd.reshape(1, -1), Wa, ba.reshape(1, -1),
        Wq[:, :HD], bq[:HD].reshape(1, -1), Wq[:, HD:], bq[HD:].reshape(1, -1),
        Wk[:, :HD], bk[:HD].reshape(1, -1), Wk[:, HD:], bk[HD:].reshape(1, -1),
        Wv[:, :HD], bv[:HD].reshape(1, -1), Wv[:, HD:], bv[HD:].reshape(1, -1),
        Wo[:HD], Wo[HD:], bo.reshape(1, -1),
        Wc1[:ZW], Wc1[ZW:], bc1.reshape(1, -1), Wc2, bc2.reshape(1, -1),
        Wu1[:ZW], Wu1[ZW:], bu1.reshape(1, -1), Wu2, bu2.reshape(1, -1),
    ]
    log_c, log_u, x_hat, out_z = _gnn_call(wmat3, xg96, snps_feat,
                                           weights, 64)
    return (log_c, log_u, x_hat, out_z)


# speedup vs baseline: 1.2405x; 1.0015x over previous
import functools

import jax
import jax.numpy as jnp
from jax import lax
from jax.experimental import pallas as pl
from jax.experimental.pallas import tpu as pltpu
from jax.experimental.pallas import tpu_sc as plsc

B = 512
ROIS = 90
RP = 96
EPG = 2880
E = B * EPG
DIM = 96
NH = 2
HD = 48
ATTEN_S = 20
ZW = ROIS * DIM
NC = 2
NS = 16
NW = NC * NS
GPW = B // NW


def _adj_body(row_hbm, col_hbm, ew_hbm, out_hbm,
              r0, r1, c0, c1, w0, w1, acc0, acc1, se0, se1, so0, so1):
    wid = lax.axis_index("s") * NC + lax.axis_index("c")
    g0 = wid * GPW

    def load_edges(g, rb, cb, wb, se):
        sl = pl.ds(g * EPG, EPG)
        pltpu.async_copy(row_hbm.at[sl], rb, se)
        pltpu.async_copy(col_hbm.at[sl], cb, se)
        pltpu.async_copy(ew_hbm.at[sl], wb, se)

    def wait_edges(rb, cb, wb, se):
        sl = pl.ds(0, EPG)
        pltpu.make_async_copy(row_hbm.at[sl], rb, se).wait()
        pltpu.make_async_copy(col_hbm.at[sl], cb, se).wait()
        pltpu.make_async_copy(ew_hbm.at[sl], wb, se).wait()

    load_edges(g0, r0, c0, w0, se0)
    load_edges(g0 + 1, r1, c1, w1, se1)
    lanes = lax.iota(jnp.int32, 16)
    ones16 = jnp.ones((16,), jnp.float32)

    def process(g, p, rb, cb, wb, acc, se, so):
        @pl.when(p >= 2)
        def _():
            pltpu.make_async_copy(acc, out_hbm.at[0], so).wait()

        def zrow(r, c):
            for j in range(RP // 16):
                acc[r, pl.ds(j * 16, 16)] = jnp.zeros((16,), jnp.float32)
            return c

        lax.fori_loop(0, RP, zrow, 0, unroll=8)
        for j in range(RP // 16):
            dv = lanes + (j * 16)
            plsc.addupdate_scatter(acc, [dv, dv], ones16)
        wait_edges(rb, cb, wb, se)
        base = g * ROIS

        def edge(j, c):
            rv = rb[pl.ds(j * 16, 16)] - base
            cv = cb[pl.ds(j * 16, 16)] - base
            wv = wb[pl.ds(j * 16, 16)]
            plsc.addupdate_scatter(acc, [cv, rv], wv)
            return c

        lax.fori_loop(0, EPG // 16, edge, 0, unroll=8)
        pltpu.async_copy(acc, out_hbm.at[g], so)

        @pl.when(p < GPW - 2)
        def _():
            load_edges(g + 2, rb, cb, wb, se)

    def pair(p, c):
        gA = g0 + 2 * p
        process(gA, 2 * p, r0, c0, w0, acc0, se0, so0)
        process(gA + 1, 2 * p + 1, r1, c1, w1, acc1, se1, so1)
        return c

    lax.fori_loop(0, GPW // 2, pair, 0)
    pltpu.make_async_copy(acc0, out_hbm.at[0], so0).wait()
    pltpu.make_async_copy(acc1, out_hbm.at[0], so1).wait()


def _build_adjacency(row, col, ew):
    mesh = plsc.VectorSubcoreMesh(core_axis_name="c", subcore_axis_name="s")
    k = functools.partial(
        pl.kernel,
        mesh=mesh,
        compiler_params=pltpu.CompilerParams(needs_layout_passes=False),
        out_type=jax.ShapeDtypeStruct((B, RP, RP), jnp.float32),
        scratch_types=[
            pltpu.VMEM((EPG,), jnp.int32),
            pltpu.VMEM((EPG,), jnp.int32),
            pltpu.VMEM((EPG,), jnp.int32),
            pltpu.VMEM((EPG,), jnp.int32),
            pltpu.VMEM((EPG,), jnp.float32),
            pltpu.VMEM((EPG,), jnp.float32),
            pltpu.VMEM((RP, RP), jnp.float32),
            pltpu.VMEM((RP, RP), jnp.float32),
            pltpu.SemaphoreType.DMA,
            pltpu.SemaphoreType.DMA,
            pltpu.SemaphoreType.DMA,
            pltpu.SemaphoreType.DMA,
        ],
    )(_adj_body)
    return k(row, col, ew)


def _mm(a, w):
    return lax.dot_general(a, w, (((a.ndim - 1,), (0,)), ((), ())),
                           preferred_element_type=jnp.float32)


def _bmm(a, b):
    return lax.dot_general(a, b, (((2,), (1,)), ((0,), (0,))),
                           preferred_element_type=jnp.float32)


def _gnn_body(w_ref, x_ref, snps_ref,
              w1_ref, b1_ref, w2_ref, b2_ref, w3_ref, b3_ref,
              we_ref, be_ref, wd_ref, bd_ref, wa_ref, ba_ref,
              wq0_ref, bq0_ref, wq1_ref, bq1_ref,
              wk0_ref, bk0_ref, wk1_ref, bk1_ref,
              wv0_ref, bv0_ref, wv1_ref, bv1_ref,
              wo0_ref, wo1_ref, bo_ref,
              wc1a_ref, wc1b_ref, bc1_ref, wc2_ref, bc2_ref,
              wu1a_ref, wu1b_ref, bu1_ref, wu2_ref, bu2_ref,
              lc_ref, lu_ref, xhat_ref, z_ref):
    G = w_ref.shape[0]
    A = w_ref[...]
    onesRP = jnp.ones((RP, 1), jnp.float32)
    deg3 = _mm(A, onesRP)
    dinv3 = jnp.where(deg3 > 0, lax.rsqrt(deg3), 0.0)

    xg = x_ref[...]
    h0 = xg[:, :, None] * w1_ref[...][0][None, None, :]
    x1 = jnp.maximum(dinv3 * _bmm(A, dinv3 * h0) + b1_ref[...][0], 0.0)
    x2 = jnp.maximum(dinv3 * _bmm(A, dinv3 * _mm(x1, w2_ref[...]))
                     + b2_ref[...][0], 0.0)
    x3 = jnp.maximum(dinv3 * _bmm(A, dinv3 * _mm(x2, w3_ref[...]))
                     + b3_ref[...][0], 0.0)
    xcat = jnp.concatenate([x1, x2, x3], axis=2)

    snps = snps_ref[...]
    latent = jnp.tanh(_mm(snps, we_ref[...]) + be_ref[...][0])
    xhat_ref[...] = _mm(latent, wd_ref[...]) + bd_ref[...][0]

    ao = (_mm(snps, wa_ref[...]) + ba_ref[...][0]).reshape(G, ATTEN_S, DIM)

    scale = 1.0 / (HD ** 0.5)
    onesS = jnp.ones((ATTEN_S, 1), jnp.float32)
    head_w = ((wq0_ref, bq0_ref, wk0_ref, bk0_ref, wv0_ref, bv0_ref),
              (wq1_ref, bq1_ref, wk1_ref, bk1_ref, wv1_ref, bv1_ref))
    outs = []
    for wq, bq, wk, bk, wv, bv in head_w:
        qh = _mm(xcat, wq[...]) + bq[...][0]
        kh = _mm(ao, wk[...]) + bk[...][0]
        vh = _mm(ao, wv[...]) + bv[...][0]
        s = lax.dot_general(qh, kh, (((2,), (2,)), ((0,), (0,))),
                            preferred_element_type=jnp.float32) * scale
        m = jnp.max(s, axis=2, keepdims=True)
        p = jnp.exp(s - m)
        a = p * (1.0 / _mm(p, onesS))
        outs.append(_bmm(a, vh))
    attn = jnp.maximum(_mm(outs[0], wo0_ref[...]) + _mm(outs[1], wo1_ref[...])
                       + bo_ref[...][0], 0.0)
    y = ((xcat + attn) * 0.5)[:, :ROIS, :]
    yflat = y.reshape(G, ZW)
    z_ref[:, :ZW] = yflat
    z_ref[:, ZW:] = latent

    def lsm(v):
        mm_ = jnp.max(v, axis=-1, keepdims=True)
        ee = v - mm_
        return ee - jnp.log(jnp.sum(jnp.exp(ee), axis=-1, keepdims=True))

    hc = jnp.maximum(_mm(yflat, wc1a_ref[...]) + _mm(latent, wc1b_ref[...])
                     + bc1_ref[...][0], 0.0)
    lc_ref[...] = lsm(_mm(hc, wc2_ref[...]) + bc2_ref[...][0])
    hu = jnp.maximum(_mm(yflat, wu1a_ref[...]) + _mm(latent, wu1b_ref[...])
                     + bu1_ref[...][0], 0.0)
    lu_ref[...] = lsm(_mm(hu, wu2_ref[...]) + bu2_ref[...][0])


def _const_spec(arr):
    nd = arr.ndim
    return pl.BlockSpec(arr.shape, lambda i, _n=nd: (0,) * _n)


def _gnn_call(wmat3, xg96, snps, weights, G):
    grid = (B // G,)
    in_specs = [
        pl.BlockSpec((G, RP, RP), lambda i: (i, 0, 0)),
        pl.BlockSpec((G, RP), lambda i: (i, 0)),
        pl.BlockSpec((G, snps.shape[1]), lambda i: (i, 0)),
    ] + [_const_spec(w) for w in weights]
    out_specs = [
        pl.BlockSpec((G, 3), lambda i: (i, 0)),
        pl.BlockSpec((G, 2), lambda i: (i, 0)),
        pl.BlockSpec((G, 54), lambda i: (i, 0)),
        pl.BlockSpec((G, ZW + 64), lambda i: (i, 0)),
    ]
    out_shape = [
        jax.ShapeDtypeStruct((B, 3), jnp.float32),
        jax.ShapeDtypeStruct((B, 2), jnp.float32),
        jax.ShapeDtypeStruct((B, 54), jnp.float32),
        jax.ShapeDtypeStruct((B, ZW + 64), jnp.float32),
    ]
    return pl.pallas_call(
        _gnn_body, grid=grid, in_specs=in_specs, out_specs=out_specs,
        out_shape=out_shape,
    )(wmat3, xg96, snps, *weights)


def kernel(x, edge_index, edge_weight, batch, snps_feat, temperature,
           W1, b1, W2, b2, W3, b3, We, be, Wd, bd, Wa, ba, Wq, bq,
           Wk, bk, Wv, bv, Wo, bo, Wc1, bc1, Wc2, bc2, Wu1, bu1, Wu2, bu2):
    wmat3 = _build_adjacency(edge_index[0], edge_index[1], edge_weight)

    xg96 = jnp.pad(x.reshape(B, ROIS), ((0, 0), (0, RP - ROIS)))
    weights = [
        W1, b1.reshape(1, -1), W2, b2.reshape(1, -1), W3, b3.reshape(1, -1),
        We, be.reshape(1, -1), Wd, bd.reshape(1, -1), Wa, ba.reshape(1, -1),
        Wq[:, :HD], bq[:HD].reshape(1, -1), Wq[:, HD:], bq[HD:].reshape(1, -1),
        Wk[:, :HD], bk[:HD].reshape(1, -1), Wk[:, HD:], bk[HD:].reshape(1, -1),
        Wv[:, :HD], bv[:HD].reshape(1, -1), Wv[:, HD:], bv[HD:].reshape(1, -1),
        Wo[:HD], Wo[HD:], bo.reshape(1, -1),
        Wc1[:ZW], Wc1[ZW:], bc1.reshape(1, -1), Wc2, bc2.reshape(1, -1),
        Wu1[:ZW], Wu1[ZW:], bu1.reshape(1, -1), Wu2, bu2.reshape(1, -1),
    ]
    log_c, log_u, x_hat, out_z = _gnn_call(wmat3, xg96, snps_feat,
                                           weights, 64)
    return (log_c, log_u, x_hat, out_z)
